# baseline (speedup 1.0000x reference)
import functools

import jax
import jax.numpy as jnp
from jax import lax
from jax.experimental import pallas as pl
from jax.experimental.pallas import tpu as pltpu
from jax.experimental.pallas import tpu_sc as plsc

F32 = jnp.float32
NC = 2
NS = 16
NW = NC * NS
C = 80
W = 4


def _mesh():
    return plsc.VectorSubcoreMesh(core_axis_name="c", subcore_axis_name="s")


def _cparams(D):
    if D % 128 == 0:
        return None
    return pltpu.CompilerParams(use_tc_tiling_on_sc=False)


@functools.lru_cache(maxsize=None)
def _make_gather(V, D, B):
    per_w = B // NW
    nch = per_w // C
    nrounds = nch // W
    tail = nch - nrounds * W

    scratch = ([pltpu.VMEM((per_w,), jnp.int32)]
               + [pltpu.VMEM((C, D), F32) for _ in range(W)]
               + [pltpu.SemaphoreType.DMA for _ in range(2 * W)])

    @functools.partial(
        pl.kernel,
        out_type=jax.ShapeDtypeStruct((B, D), F32),
        mesh=_mesh(),
        scratch_types=scratch,
        compiler_params=_cparams(D),
    )
    def k(table, idx_hbm, out, idx_all, r0, r1, r2, r3,
          g0, g1, g2, g3, o0, o1, o2, o3):
        rows = [r0, r1, r2, r3]
        gs = [g0, g1, g2, g3]
        os = [o0, o1, o2, o3]
        wid = lax.axis_index("s") * NC + lax.axis_index("c")
        base = wid * per_w
        pltpu.sync_copy(idx_hbm.at[pl.ds(base, per_w)], idx_all)

        if nrounds > 0:
            def round_body(r, _):
                c0 = r * W
                gd = [pltpu.async_copy(
                    table.at[idx_all.at[pl.ds((c0 + w) * C, C)]],
                    rows[w], gs[w]) for w in range(W)]
                od = []
                for w in range(W):
                    gd[w].wait()
                    od.append(pltpu.async_copy(
                        rows[w], out.at[pl.ds(base + (c0 + w) * C, C)],
                        os[w]))
                for w in range(W):
                    od[w].wait()
                return 0
            lax.fori_loop(0, nrounds, round_body, 0)
        for t in range(tail):
            c = nrounds * W + t
            pltpu.async_copy(table.at[idx_all.at[pl.ds(c * C, C)]],
                             rows[0], gs[0]).wait()
            pltpu.sync_copy(rows[0], out.at[pl.ds(base + c * C, C)])

    return k


_DT = {'f32': jnp.float32, 'bf16': jnp.bfloat16}


@functools.lru_cache(maxsize=None)
def _make_multi_gather(tables, n_idx, outs, B):
    per_w = B // NW
    nch = per_w // C
    K = len(outs)
    buf_bytes = sum(C * d * (2 if t == 'bf16' else 4)
                    for (_, _, d, t) in outs)
    weff = max(1, min(4, (360 * 1024) // buf_bytes))
    weff = min(weff, nch)
    nrounds = nch // weff
    tail = nch - nrounds * weff

    scratch = ([pltpu.VMEM((per_w,), jnp.int32) for _ in range(n_idx)]
               + [pltpu.VMEM((C, d), _DT[t])
                  for _ in range(weff) for (_, _, d, t) in outs]
               + [pltpu.SemaphoreType.DMA for _ in range(2 * weff)])

    @functools.partial(
        pl.kernel,
        out_type=tuple(jax.ShapeDtypeStruct((B, d), _DT[t])
                       for (_, _, d, t) in outs),
        mesh=_mesh(),
        scratch_types=scratch,
        compiler_params=_cparams(16),
    )
    def k(*refs):
        tabs = list(refs[:len(tables)])
        idx_hbm = list(refs[len(tables):len(tables) + n_idx])
        o = len(tables) + n_idx
        out_hbm = list(refs[o:o + K])
        o += K
        idx_all = list(refs[o:o + n_idx])
        o += n_idx
        bufs = [list(refs[o + w * K:o + (w + 1) * K]) for w in range(weff)]
        o += weff * K
        gsem = list(refs[o:o + weff])
        osem = list(refs[o + weff:o + 2 * weff])

        wid = lax.axis_index("s") * NC + lax.axis_index("c")
        base = wid * per_w
        for i in range(n_idx):
            pltpu.sync_copy(idx_hbm[i].at[pl.ds(base, per_w)], idx_all[i])

        def do_block(c0, nw):
            gd = []
            for w in range(nw):
                for kk, (tn, ii, d, t) in enumerate(outs):
                    gd.append(pltpu.async_copy(
                        tabs[tn].at[idx_all[ii].at[pl.ds((c0 + w) * C, C)]],
                        bufs[w][kk], gsem[w]))
            od = []
            for w in range(nw):
                for kk in range(K):
                    gd[w * K + kk].wait()
                    od.append(pltpu.async_copy(
                        bufs[w][kk],
                        out_hbm[kk].at[pl.ds(base + (c0 + w) * C, C)],
                        osem[w]))
            for d_ in od:
                d_.wait()

        if nrounds > 0:
            def round_body(r, _):
                do_block(r * weff, weff)
                return 0
            lax.fori_loop(0, nrounds, round_body, 0)
        if tail:
            do_block(nrounds * weff, tail)

    return k


@functools.lru_cache(maxsize=None)
def _make_conv(E, V, N_pad):
    DH = 64
    per_t = E // NS
    nch = per_t // C
    nrounds = nch // W
    tail = nch - nrounds * W
    rows_ps = N_pad // NS
    reps = rows_ps // C

    scratch = ([pltpu.VMEM((per_t,), jnp.int32)]
               + [pltpu.VMEM((C, DH), F32) for _ in range(W)]
               + [pltpu.VMEM((C,), jnp.int32) for _ in range(W)]
               + [pltpu.VMEM((C, DH), F32)]
               + [pltpu.VMEM_SHARED((N_pad, DH), F32)]
               + [pltpu.SemaphoreType.DMA for _ in range(2 * W)])

    @functools.partial(
        pl.kernel,
        out_type=jax.ShapeDtypeStruct((N_pad, 128), F32),
        mesh=_mesh(),
        scratch_types=scratch,
        compiler_params=_cparams(DH),
    )
    def k(tabA, tabB, gi_hbm, gj_hbm, out, gi_all, v0, v1, v2, v3,
          i0, i1, i2, i3, zbuf, acc, s0, s1, s2, s3, t0, t1, t2, t3):
        vb = [v0, v1, v2, v3]
        ib = [i0, i1, i2, i3]
        vs = [s0, s1, s2, s3]
        isem = [t0, t1, t2, t3]
        cid = lax.axis_index("c")
        sid = lax.axis_index("s")
        col0 = cid * DH
        dpg = DH // 16

        def zb(t, _):
            zbuf[t // dpg, pl.ds((t % dpg) * 16, 16)] = jnp.zeros((16,), F32)
            return 0
        lax.fori_loop(0, C * dpg, zb, 0)

        def zc(r, _):
            pltpu.sync_copy(zbuf, acc.at[pl.ds(sid * rows_ps + r * C, C)])
            return 0
        lax.fori_loop(0, reps, zc, 0)
        plsc.subcore_barrier()

        base = sid * per_t
        pltpu.sync_copy(gi_hbm.at[pl.ds(base, per_t)], gi_all)

        def do_block(tab, c0, nw):
            gd = [pltpu.async_copy(
                tab.at[gi_all.at[pl.ds((c0 + w) * C, C)]],
                vb[w], vs[w]) for w in range(nw)]
            idd = [pltpu.async_copy(
                gj_hbm.at[pl.ds(base + (c0 + w) * C, C)],
                ib[w], isem[w]) for w in range(nw)]
            for w in range(nw):
                gd[w].wait()
                idd[w].wait()
                pltpu.sync_copy(vb[w], acc.at[ib[w]], add=True)

        def run(tab):
            if nrounds > 0:
                def round_body(r, _):
                    do_block(tab, r * W, W)
                    return 0
                lax.fori_loop(0, nrounds, round_body, 0)
            if tail:
                do_block(tab, nrounds * W, tail)

        @pl.when(cid == 0)
        def _():
            run(tabA)

        @pl.when(cid == 1)
        def _():
            run(tabB)

        plsc.subcore_barrier()
        pltpu.sync_copy(acc.at[pl.ds(sid * rows_ps, rows_ps)],
                        out.at[pl.ds(sid * rows_ps, rows_ps),
                               pl.ds(col0, DH)])

    return k


@functools.lru_cache(maxsize=None)
def _make_conv_narrow(E, V, N_pad):
    D = 16
    per_w = E // NW
    nch = per_w // C
    nrounds = nch // W
    tail = nch - nrounds * W
    rows_ps = N_pad // NS
    reps = rows_ps // C

    scratch = ([pltpu.VMEM((per_w,), jnp.int32)]
               + [pltpu.VMEM((C, D), F32) for _ in range(W)]
               + [pltpu.VMEM((C,), jnp.int32) for _ in range(W)]
               + [pltpu.VMEM((C, D), F32)]
               + [pltpu.VMEM_SHARED((N_pad, D), F32)]
               + [pltpu.SemaphoreType.DMA for _ in range(2 * W)])

    @functools.partial(
        pl.kernel,
        out_type=jax.ShapeDtypeStruct((NC * N_pad, D), F32),
        mesh=_mesh(),
        scratch_types=scratch,
        compiler_params=_cparams(D),
    )
    def k(table, gi_hbm, gj_hbm, out, gi_all, v0, v1, v2, v3,
          i0, i1, i2, i3, zbuf, acc, s0, s1, s2, s3, t0, t1, t2, t3):
        vb = [v0, v1, v2, v3]
        ib = [i0, i1, i2, i3]
        vs = [s0, s1, s2, s3]
        isem = [t0, t1, t2, t3]
        cid = lax.axis_index("c")
        sid = lax.axis_index("s")
        wid = sid * NC + cid

        def zb(t, _):
            zbuf[t, pl.ds(0, 16)] = jnp.zeros((16,), F32)
            return 0
        lax.fori_loop(0, C, zb, 0)

        def zc(r, _):
            pltpu.sync_copy(zbuf, acc.at[pl.ds(sid * rows_ps + r * C, C)])
            return 0
        lax.fori_loop(0, reps, zc, 0)
        plsc.subcore_barrier()

        base = wid * per_w
        pltpu.sync_copy(gi_hbm.at[pl.ds(base, per_w)], gi_all)

        def do_block(c0, nw):
            gd = [pltpu.async_copy(
                table.at[gi_all.at[pl.ds((c0 + w) * C, C)]],
                vb[w], vs[w]) for w in range(nw)]
            idd = [pltpu.async_copy(
                gj_hbm.at[pl.ds(base + (c0 + w) * C, C)],
                ib[w], isem[w]) for w in range(nw)]
            for w in range(nw):
                gd[w].wait()
                idd[w].wait()
                pltpu.sync_copy(vb[w], acc.at[ib[w]], add=True)

        if nrounds > 0:
            def round_body(r, _):
                do_block(r * W, W)
                return 0
            lax.fori_loop(0, nrounds, round_body, 0)
        if tail:
            do_block(nrounds * W, tail)

        plsc.subcore_barrier()
        pltpu.sync_copy(acc.at[pl.ds(sid * rows_ps, rows_ps)],
                        out.at[pl.ds(cid * N_pad + sid * rows_ps, rows_ps)])

    return k


@functools.lru_cache(maxsize=None)
def _make_scatter128(E, N_pad):
    DH = 64
    per_t = E // NS
    nch = per_t // C
    nrounds = nch // W
    tail = nch - nrounds * W
    rows_ps = N_pad // NS
    reps = rows_ps // C

    scratch = ([pltpu.VMEM((C, DH), F32) for _ in range(W)]
               + [pltpu.VMEM((C,), jnp.int32) for _ in range(W)]
               + [pltpu.VMEM((C, DH), F32)]
               + [pltpu.VMEM_SHARED((N_pad, DH), F32)]
               + [pltpu.SemaphoreType.DMA for _ in range(2 * W)])

    @functools.partial(
        pl.kernel,
        out_type=jax.ShapeDtypeStruct((N_pad, 128), F32),
        mesh=_mesh(),
        scratch_types=scratch,
        compiler_params=_cparams(DH),
    )
    def k(vals_hbm, idx_hbm, out, v0, v1, v2, v3, i0, i1, i2, i3,
          zbuf, acc, s0, s1, s2, s3, t0, t1, t2, t3):
        vb = [v0, v1, v2, v3]
        ib = [i0, i1, i2, i3]
        vs = [s0, s1, s2, s3]
        isem = [t0, t1, t2, t3]
        cid = lax.axis_index("c")
        sid = lax.axis_index("s")
        col0 = cid * DH
        dpg = DH // 16

        def zb(t, _):
            zbuf[t // dpg, pl.ds((t % dpg) * 16, 16)] = jnp.zeros((16,), F32)
            return 0
        lax.fori_loop(0, C * dpg, zb, 0)

        def zc(r, _):
            pltpu.sync_copy(zbuf, acc.at[pl.ds(sid * rows_ps + r * C, C)])
            return 0
        lax.fori_loop(0, reps, zc, 0)
        plsc.subcore_barrier()

        base = sid * per_t

        if nrounds > 0:
            def round_body(r, _):
                c0 = r * W
                vd = [pltpu.async_copy(
                    vals_hbm.at[pl.ds(base + (c0 + w) * C, C),
                                pl.ds(col0, DH)],
                    vb[w], vs[w]) for w in range(W)]
                idd = [pltpu.async_copy(
                    idx_hbm.at[pl.ds(base + (c0 + w) * C, C)],
                    ib[w], isem[w]) for w in range(W)]
                for w in range(W):
                    vd[w].wait()
                    idd[w].wait()
                    pltpu.sync_copy(vb[w], acc.at[ib[w]], add=True)
                return 0
            lax.fori_loop(0, nrounds, round_body, 0)
        for t in range(tail):
            c = nrounds * W + t
            pltpu.sync_copy(
                vals_hbm.at[pl.ds(base + c * C, C), pl.ds(col0, DH)], vb[0])
            pltpu.sync_copy(idx_hbm.at[pl.ds(base + c * C, C)], ib[0])
            pltpu.sync_copy(vb[0], acc.at[ib[0]], add=True)

        plsc.subcore_barrier()
        pltpu.sync_copy(acc.at[pl.ds(sid * rows_ps, rows_ps)],
                        out.at[pl.ds(sid * rows_ps, rows_ps),
                               pl.ds(col0, DH)])

    return k


@functools.lru_cache(maxsize=None)
def _make_scatter_narrow(E, N_pad, D):
    per_w = E // NW
    nch = per_w // C
    nrounds = nch // W
    tail = nch - nrounds * W
    rows_ps = N_pad // NS
    reps = rows_ps // C

    scratch = ([pltpu.VMEM((C, D), F32) for _ in range(W)]
               + [pltpu.VMEM((C,), jnp.int32) for _ in range(W)]
               + [pltpu.VMEM((C, D), F32)]
               + [pltpu.VMEM_SHARED((N_pad, D), F32)]
               + [pltpu.SemaphoreType.DMA for _ in range(2 * W)])

    @functools.partial(
        pl.kernel,
        out_type=jax.ShapeDtypeStruct((NC * N_pad, D), F32),
        mesh=_mesh(),
        scratch_types=scratch,
        compiler_params=_cparams(D),
    )
    def k(vals_hbm, idx_hbm, out, v0, v1, v2, v3, i0, i1, i2, i3,
          zbuf, acc, s0, s1, s2, s3, t0, t1, t2, t3):
        vb = [v0, v1, v2, v3]
        ib = [i0, i1, i2, i3]
        vs = [s0, s1, s2, s3]
        isem = [t0, t1, t2, t3]
        cid = lax.axis_index("c")
        sid = lax.axis_index("s")
        wid = sid * NC + cid
        dpg = D // 16

        def zb(t, _):
            zbuf[t // dpg, pl.ds((t % dpg) * 16, 16)] = jnp.zeros((16,), F32)
            return 0
        lax.fori_loop(0, C * dpg, zb, 0)

        def zc(r, _):
            pltpu.sync_copy(zbuf, acc.at[pl.ds(sid * rows_ps + r * C, C)])
            return 0
        lax.fori_loop(0, reps, zc, 0)
        plsc.subcore_barrier()

        base = wid * per_w

        if nrounds > 0:
            def round_body(r, _):
                c0 = r * W
                vd = [pltpu.async_copy(
                    vals_hbm.at[pl.ds(base + (c0 + w) * C, C)],
                    vb[w], vs[w]) for w in range(W)]
                idd = [pltpu.async_copy(
                    idx_hbm.at[pl.ds(base + (c0 + w) * C, C)],
                    ib[w], isem[w]) for w in range(W)]
                for w in range(W):
                    vd[w].wait()
                    idd[w].wait()
                    pltpu.sync_copy(vb[w], acc.at[ib[w]], add=True)
                return 0
            lax.fori_loop(0, nrounds, round_body, 0)
        for t in range(tail):
            c = nrounds * W + t
            pltpu.sync_copy(vals_hbm.at[pl.ds(base + c * C, C)], vb[0])
            pltpu.sync_copy(idx_hbm.at[pl.ds(base + c * C, C)], ib[0])
            pltpu.sync_copy(vb[0], acc.at[ib[0]], add=True)

        plsc.subcore_barrier()
        pltpu.sync_copy(acc.at[pl.ds(sid * rows_ps, rows_ps)],
                        out.at[pl.ds(cid * N_pad + sid * rows_ps, rows_ps)])

    return k


@functools.lru_cache(maxsize=None)
def _make_stats(E, N_pad):
    EC = 2000
    per_s = E // NS
    nch = per_s // EC
    rows_ps = N_pad // NS
    nvec = N_pad // 16
    nv = rows_ps // 16

    scratch = [
        pltpu.VMEM((N_pad,), F32),
        pltpu.VMEM((N_pad,), F32),
        pltpu.VMEM((EC,), jnp.int32),
        pltpu.VMEM((EC,), jnp.int32),
        pltpu.VMEM((NS, rows_ps), F32),
        pltpu.VMEM((rows_ps,), F32),
        pltpu.VMEM_SHARED((NS, N_pad), F32),
        pltpu.VMEM_SHARED((N_pad,), F32),
    ]

    @functools.partial(
        pl.kernel,
        out_type=(jax.ShapeDtypeStruct((N_pad,), F32),
                  jax.ShapeDtypeStruct((N_pad,), F32)),
        mesh=_mesh(),
        scratch_types=scratch,
        compiler_params=pltpu.CompilerParams(use_tc_tiling_on_sc=False,
                                             needs_layout_passes=False),
    )
    def k(ihbm, jhbm, invdeg_out, invagg_out,
          acc_t, inv_t, ib, jb, colbuf, slice_b, stage, degs):
        cid = lax.axis_index("c")
        sid = lax.axis_index("s")
        ones = jnp.ones((16,), F32)
        r0 = sid * rows_ps

        def zero_acc(t, _):
            acc_t[pl.ds(t * 16, 16)] = jnp.zeros((16,), F32)
            return 0

        def combine_to_slice():
            pltpu.sync_copy(acc_t, stage.at[sid])
            plsc.subcore_barrier()
            pltpu.sync_copy(stage.at[:, pl.ds(r0, rows_ps)], colbuf)

            def comb(v, _):
                s = jnp.zeros((16,), F32)
                for t in range(NS):
                    s = s + colbuf[t, pl.ds(v * 16, 16)]
                slice_b[pl.ds(v * 16, 16)] = s
                return 0
            lax.fori_loop(0, nv, comb, 0)

        lax.fori_loop(0, nvec, zero_acc, 0)

        def ch1(c, _):
            pltpu.sync_copy(ihbm.at[pl.ds(sid * per_s + c * EC, EC)], ib)

            def v1(v, _):
                iv = ib[pl.ds(v * 16, 16)]
                plsc.addupdate_scatter(acc_t, [iv], ones)
                return 0
            lax.fori_loop(0, EC // 16, v1, 0)
            return 0
        lax.fori_loop(0, nch, ch1, 0)

        combine_to_slice()
        pltpu.sync_copy(slice_b, degs.at[pl.ds(r0, rows_ps)])

        def invv(v, _):
            slice_b[pl.ds(v * 16, 16)] = 1.0 / slice_b[pl.ds(v * 16, 16)]
            return 0
        lax.fori_loop(0, nv, invv, 0)

        @pl.when(cid == 0)
        def _():
            pltpu.sync_copy(slice_b, invdeg_out.at[pl.ds(r0, rows_ps)])

        plsc.subcore_barrier()
        pltpu.sync_copy(degs, inv_t)

        def inv2(v, _):
            inv_t[pl.ds(v * 16, 16)] = 1.0 / inv_t[pl.ds(v * 16, 16)]
            return 0
        lax.fori_loop(0, nvec, inv2, 0)

        lax.fori_loop(0, nvec, zero_acc, 0)

        def ch2(c, _):
            pltpu.sync_copy(ihbm.at[pl.ds(sid * per_s + c * EC, EC)], ib)
            pltpu.sync_copy(jhbm.at[pl.ds(sid * per_s + c * EC, EC)], jb)

            def v2(v, _):
                iv = ib[pl.ds(v * 16, 16)]
                jv = jb[pl.ds(v * 16, 16)]
                g = plsc.load_gather(inv_t, [iv])
                plsc.addupdate_scatter(acc_t, [jv], g)
                return 0
            lax.fori_loop(0, EC // 16, v2, 0)
            return 0
        lax.fori_loop(0, nch, ch2, 0)

        combine_to_slice()

        def inva(v, _):
            slice_b[pl.ds(v * 16, 16)] = 1.0 / (slice_b[pl.ds(v * 16, 16)]
                                                + 1e-12)
            return 0
        lax.fori_loop(0, nv, inva, 0)

        @pl.when(cid == 0)
        def _():
            pltpu.sync_copy(slice_b, invagg_out.at[pl.ds(r0, rows_ps)])

    return k


def _full(shape):
    return pl.BlockSpec(shape, lambda i: tuple(0 for _ in shape))


def _rows(bs, d):
    return pl.BlockSpec((bs, d), lambda i: (i, 0))


def _edge_mlp(Pi, Qj, posi, posj, W0g, b0, W1, b1, W2, b2, g, beta):
    E = Pi.shape[0]
    BE = 512
    grid = E // BE

    def body(pi, qj, pa, pb, w0g, b0r, w1, b1r, w2, b2r, gr, br, out):
        d = pa[...] - pb[...]
        nrm = jnp.sqrt(jnp.sum(d * d, axis=1, keepdims=True))
        sel = (lax.broadcasted_iota(jnp.int32, (1, 16), 1) == 3).astype(F32)
        ge = d + nrm * sel
        s = pi[...].astype(F32) + qj[...].astype(F32)
        h0 = jnp.maximum(s + ge @ w0g[...] + b0r[...], 0.0)
        h1 = jnp.maximum(h0 @ w1[...] + b1r[...], 0.0)
        h2 = h1 @ w2[...] + b2r[...]
        mu = jnp.mean(h2, axis=1, keepdims=True)
        xc = h2 - mu
        var = jnp.mean(xc * xc, axis=1, keepdims=True)
        out[...] = xc / jnp.sqrt(var + 1e-5) * gr[...] + br[...]

    return pl.pallas_call(
        body,
        grid=(grid,),
        in_specs=[_rows(BE, 128), _rows(BE, 128), _rows(BE, 16),
                  _rows(BE, 16), _full((16, 128)), _full((1, 128)),
                  _full((128, 128)), _full((1, 128)), _full((128, 128)),
                  _full((1, 128)), _full((1, 128)), _full((1, 128))],
        out_specs=_rows(BE, 128),
        out_shape=jax.ShapeDtypeStruct((E, 128), F32),
    )(Pi, Qj, posi, posj, W0g, b0, W1, b1, W2, b2, g, beta)


def _node_mlp(x, aggr, A0, B0, b0, W1, b1, W2, b2, g, beta, res2=None):
    N = x.shape[0]
    BN = 2000 if N % 2000 == 0 else 512
    grid = N // BN

    def body(*refs):
        if res2 is None:
            (xr, ar, a0, b0m, b0r, w1, b1r, w2, b2r, gr, br, out) = refs
        else:
            (xr, ar, a0, b0m, b0r, w1, b1r, w2, b2r, gr, br, r2,
             out) = refs
        h0 = jnp.maximum(xr[...] @ a0[...] + ar[...] @ b0m[...] + b0r[...],
                         0.0)
        h1 = jnp.maximum(h0 @ w1[...] + b1r[...], 0.0)
        h2 = h1 @ w2[...] + b2r[...]
        mu = jnp.mean(h2, axis=1, keepdims=True)
        xc = h2 - mu
        var = jnp.mean(xc * xc, axis=1, keepdims=True)
        o = xc / jnp.sqrt(var + 1e-5) * gr[...] + br[...] + xr[...]
        if res2 is not None:
            o = o + r2[...]
        out[...] = o

    ins = [x, aggr, A0, B0, b0, W1, b1, W2, b2, g, beta]
    specs = [_rows(BN, 128), _rows(BN, 128),
             _full((128, 128)), _full((128, 128)), _full((1, 128)),
             _full((128, 128)), _full((1, 128)), _full((128, 128)),
             _full((1, 128)), _full((1, 128)), _full((1, 128))]
    if res2 is not None:
        ins.append(res2)
        specs.append(_rows(BN, 128))
    return pl.pallas_call(
        body,
        grid=(grid,),
        in_specs=specs,
        out_specs=_rows(BN, 128),
        out_shape=jax.ShapeDtypeStruct((N, 128), F32),
    )(*ins)


def _matmul2(x, Wa, Wb):
    N = x.shape[0]
    BN = 2000 if N % 2000 == 0 else 512
    grid = N // BN

    def body(xr, wa, wb, oa, ob):
        oa[...] = (xr[...] @ wa[...]).astype(jnp.bfloat16)
        ob[...] = (xr[...] @ wb[...]).astype(jnp.bfloat16)

    return pl.pallas_call(
        body,
        grid=(grid,),
        in_specs=[_rows(BN, 128), _full((128, 128)), _full((128, 128))],
        out_specs=[_rows(BN, 128), _rows(BN, 128)],
        out_shape=[jax.ShapeDtypeStruct((N, 128), jnp.bfloat16),
                   jax.ShapeDtypeStruct((N, 128), jnp.bfloat16)],
    )(x, Wa, Wb)


def _combine_scale(p0, p1, s, N, D):
    BN = 2000 if N % 2000 == 0 else 512
    grid = N // BN

    def body(a, b, sr, out):
        out[...] = (a[...] + b[...]) * sr[...]

    return pl.pallas_call(
        body,
        grid=(grid,),
        in_specs=[_rows(BN, D), _rows(BN, D), _rows(BN, 1)],
        out_specs=_rows(BN, D),
        out_shape=jax.ShapeDtypeStruct((N, D), F32),
    )(p0, p1, s)


def _scale(x, s):
    N, D = x.shape
    BN = 2000 if N % 2000 == 0 else 512
    grid = N // BN

    def body(a, sr, out):
        out[...] = a[...] * sr[...]

    return pl.pallas_call(
        body,
        grid=(grid,),
        in_specs=[_rows(BN, D), _rows(BN, 1)],
        out_specs=_rows(BN, D),
        out_shape=jax.ShapeDtypeStruct((N, D), F32),
    )(x, s)


def _scale_split(x, s):
    N, D = x.shape
    BN = 2000 if N % 2000 == 0 else 512
    grid = N // BN

    def body(a, sr, oa, ob):
        v = a[...] * sr[...]
        oa[...] = v[:, :64]
        ob[...] = v[:, 64:]

    return pl.pallas_call(
        body,
        grid=(grid,),
        in_specs=[_rows(BN, D), _rows(BN, 1)],
        out_specs=[_rows(BN, 64), _rows(BN, 64)],
        out_shape=[jax.ShapeDtypeStruct((N, 64), F32),
                   jax.ShapeDtypeStruct((N, 64), F32)],
    )(x, s)


def _split_edge_w(p):
    W0 = p['W0']
    W0g = jnp.zeros((16, 128), F32).at[:4].set(W0[:4])
    Wxi = W0[4:132]
    Wxj = W0[132:260]
    return (W0g, Wxi, Wxj, p['b0'].reshape(1, -1), p['W1'],
            p['b1'].reshape(1, -1), p['W2'], p['b2'].reshape(1, -1),
            p['g'].reshape(1, -1), p['beta'].reshape(1, -1))


def _split_node_w(p):
    W0 = p['W0']
    return (W0[:128], W0[128:], p['b0'].reshape(1, -1), p['W1'],
            p['b1'].reshape(1, -1), p['W2'], p['b2'].reshape(1, -1),
            p['g'].reshape(1, -1), p['beta'].reshape(1, -1))


def _gmp(x, posarg, gi, gj, gj_scat, pe, pn, N, E, N_pad, res2=None):
    W0g, Wxi, Wxj, b0, W1, b1, W2, b2, g, beta = _split_edge_w(pe)
    P, Q = _matmul2(x, Wxi, Wxj)
    V = x.shape[0]
    if isinstance(posarg, tuple):
        posi, posj = posarg
        mg = _make_multi_gather(((V, 128), (V, 128)), 2,
                                ((0, 0, 128, 'bf16'), (1, 1, 128, 'bf16')),
                                E)
        Pi, Qj = mg(P, Q, gi, gj)
    else:
        mg = _make_multi_gather(
            ((V, 128), (V, 128), (V, 16)), 2,
            ((0, 0, 128, 'bf16'), (1, 1, 128, 'bf16'),
             (2, 0, 16, 'f32'), (2, 1, 16, 'f32')), E)
        Pi, Qj, posi, posj = mg(P, Q, posarg, gi, gj)
    ee = _edge_mlp(Pi, Qj, posi, posj, W0g, b0, W1, b1, W2, b2, g, beta)
    aggr = _make_scatter128(E, N_pad)(ee, gj_scat)
    A0, B0, nb0, nW1, nb1, nW2, nb2, ng, nbeta = _split_node_w(pn)
    return _node_mlp(x[:N], aggr[:N], A0, B0, nb0,
                     nW1, nb1, nW2, nb2, ng, nbeta, res2=res2)


def kernel(h, pos, params, m_gs_0, m_gs_1, m_ids_0):
    N0, LD = h.shape
    E0 = m_gs_0.shape[1]
    N1 = m_ids_0.shape[0]
    E1 = m_gs_1.shape[1]
    N0p = 10240
    N1p = 2560
    E1p = 81920
    Bp = 2560

    i0 = m_gs_0[0]
    j0 = m_gs_0[1]
    pos_pad = jnp.pad(pos, ((0, 0), (0, 16 - pos.shape[1])))

    i1g = jnp.pad(m_gs_1[0], (0, E1p - E1))
    j1g = jnp.pad(m_gs_1[1], (0, E1p - E1))
    j1s = jnp.pad(m_gs_1[1], (0, E1p - E1), constant_values=N1p - 1)
    ids_g = jnp.pad(m_ids_0, (0, Bp - N1))
    ids_s = jnp.pad(m_ids_0, (0, Bp - N1), constant_values=N0p - 1)

    invdeg, invaggr = _make_stats(E0, N0p)(i0, j0)
    invdeg2 = invdeg[:N0].reshape(N0, 1)
    invaggr2 = invaggr[:N0].reshape(N0, 1)

    down = _make_multi_gather(((N0, 128), (N0, 128), (N0, 16)), 2,
                              ((0, 0, 128, 'bf16'), (1, 1, 128, 'bf16'),
                               (2, 0, 16, 'f32'), (2, 1, 16, 'f32')), E0)
    h1, posi0, posj0 = None, None, None
    W0g, Wxi, Wxj, b0, W1, b1, W2, b2, g, beta = _split_edge_w(
        params['down_e'])
    P, Q = _matmul2(h, Wxi, Wxj)
    Pi, Qj, posi0, posj0 = down(P, Q, pos_pad, i0, j0)
    ee = _edge_mlp(Pi, Qj, posi0, posj0, W0g, b0, W1, b1, W2, b2, g, beta)
    aggr = _make_scatter128(E0, N0p)(ee, j0)
    A0, B0, nb0, nW1, nb1, nW2, nb2, ng, nbeta = _split_node_w(
        params['down_n'])
    h1 = _node_mlp(h, aggr[:N0], A0, B0, nb0, nW1, nb1, nW2, nb2, ng, nbeta)

    conv0 = _make_conv(E0, N0, N0p)
    hhA, hhB = _scale_split(h1, invdeg2)
    h_conv = _scale(conv0(hhA, hhB, i0, j0)[:N0], invaggr2)

    pscaled = _scale(pos_pad, invdeg2)
    pparts = _make_conv_narrow(E0, N0, N0p)(pscaled, i0, j0)
    pos_conv = _combine_scale(pparts[:N0p], pparts[N0p:], invaggr2, N0, 16)

    h_pool, pos_pool = _make_multi_gather(
        ((N0, 128), (N0, 16)), 1, ((0, 0, 128, 'f32'), (1, 0, 16, 'f32')),
        Bp)(h_conv, pos_conv, ids_g)

    h_bot = _gmp(h_pool, pos_pool, i1g, j1g, j1s, params['bot_e'],
                 params['bot_n'], N1p, E1p, N1p)

    xup = _make_scatter128(Bp, N0p)(h_bot, ids_s)
    xupA, xupB = _scale_split(xup[:N0], invaggr2)
    h_up = _scale(conv0(xupA, xupB, j0, i0)[:N0], invdeg2)

    out = _gmp(h_up, (posi0, posj0), i0, j0, j0, params['up_e'],
               params['up_n'], N0, E0, N0p, res2=h1)
    return out


# speedup vs baseline: 1.3458x; 1.3458x over previous
import functools

import jax
import jax.numpy as jnp
from jax import lax
from jax.experimental import pallas as pl
from jax.experimental.pallas import tpu as pltpu
from jax.experimental.pallas import tpu_sc as plsc

F32 = jnp.float32
NC = 2
NS = 16
NW = NC * NS
C = 80
W = 4


def _mesh():
    return plsc.VectorSubcoreMesh(core_axis_name="c", subcore_axis_name="s")


def _cparams(D):
    if D % 128 == 0:
        return None
    return pltpu.CompilerParams(use_tc_tiling_on_sc=False)


@functools.lru_cache(maxsize=None)
def _make_gather(V, D, B):
    per_w = B // NW
    nch = per_w // C
    nrounds = nch // W
    tail = nch - nrounds * W

    scratch = ([pltpu.VMEM((per_w,), jnp.int32)]
               + [pltpu.VMEM((C, D), F32) for _ in range(W)]
               + [pltpu.SemaphoreType.DMA for _ in range(2 * W)])

    @functools.partial(
        pl.kernel,
        out_type=jax.ShapeDtypeStruct((B, D), F32),
        mesh=_mesh(),
        scratch_types=scratch,
        compiler_params=_cparams(D),
    )
    def k(table, idx_hbm, out, idx_all, r0, r1, r2, r3,
          g0, g1, g2, g3, o0, o1, o2, o3):
        rows = [r0, r1, r2, r3]
        gs = [g0, g1, g2, g3]
        os = [o0, o1, o2, o3]
        wid = lax.axis_index("s") * NC + lax.axis_index("c")
        base = wid * per_w
        pltpu.sync_copy(idx_hbm.at[pl.ds(base, per_w)], idx_all)

        if nrounds > 0:
            def round_body(r, _):
                c0 = r * W
                gd = [pltpu.async_copy(
                    table.at[idx_all.at[pl.ds((c0 + w) * C, C)]],
                    rows[w], gs[w]) for w in range(W)]
                od = []
                for w in range(W):
                    gd[w].wait()
                    od.append(pltpu.async_copy(
                        rows[w], out.at[pl.ds(base + (c0 + w) * C, C)],
                        os[w]))
                for w in range(W):
                    od[w].wait()
                return 0
            lax.fori_loop(0, nrounds, round_body, 0)
        for t in range(tail):
            c = nrounds * W + t
            pltpu.async_copy(table.at[idx_all.at[pl.ds(c * C, C)]],
                             rows[0], gs[0]).wait()
            pltpu.sync_copy(rows[0], out.at[pl.ds(base + c * C, C)])

    return k


_DT = {'f32': jnp.float32, 'bf16': jnp.bfloat16}


@functools.lru_cache(maxsize=None)
def _make_multi_gather(tables, n_idx, outs, B):
    per_w = B // NW
    nch = per_w // C
    K = len(outs)
    buf_bytes = sum(C * d * (2 if t == 'bf16' else 4)
                    for (_, _, d, t) in outs)
    weff = max(1, min(4, (360 * 1024) // buf_bytes))
    weff = min(weff, nch)
    nrounds = nch // weff
    tail = nch - nrounds * weff

    scratch = ([pltpu.VMEM((per_w,), jnp.int32) for _ in range(n_idx)]
               + [pltpu.VMEM((C, d), _DT[t])
                  for _ in range(weff) for (_, _, d, t) in outs]
               + [pltpu.SemaphoreType.DMA for _ in range(2 * weff)])

    narrow = any(d % 128 != 0 for (_, _, d, t) in outs)

    @functools.partial(
        pl.kernel,
        out_type=tuple(jax.ShapeDtypeStruct((B, d), _DT[t])
                       for (_, _, d, t) in outs),
        mesh=_mesh(),
        scratch_types=scratch,
        compiler_params=_cparams(16 if narrow else 128),
    )
    def k(*refs):
        tabs = list(refs[:len(tables)])
        idx_hbm = list(refs[len(tables):len(tables) + n_idx])
        o = len(tables) + n_idx
        out_hbm = list(refs[o:o + K])
        o += K
        idx_all = list(refs[o:o + n_idx])
        o += n_idx
        bufs = [list(refs[o + w * K:o + (w + 1) * K]) for w in range(weff)]
        o += weff * K
        gsem = list(refs[o:o + weff])
        osem = list(refs[o + weff:o + 2 * weff])

        wid = lax.axis_index("s") * NC + lax.axis_index("c")
        base = wid * per_w
        for i in range(n_idx):
            pltpu.sync_copy(idx_hbm[i].at[pl.ds(base, per_w)], idx_all[i])

        def do_block(c0, nw):
            gd = []
            for w in range(nw):
                for kk, (tn, ii, d, t) in enumerate(outs):
                    gd.append(pltpu.async_copy(
                        tabs[tn].at[idx_all[ii].at[pl.ds((c0 + w) * C, C)]],
                        bufs[w][kk], gsem[w]))
            od = []
            for w in range(nw):
                for kk in range(K):
                    gd[w * K + kk].wait()
                    od.append(pltpu.async_copy(
                        bufs[w][kk],
                        out_hbm[kk].at[pl.ds(base + (c0 + w) * C, C)],
                        osem[w]))
            for d_ in od:
                d_.wait()

        if nrounds > 0:
            def round_body(r, _):
                do_block(r * weff, weff)
                return 0
            lax.fori_loop(0, nrounds, round_body, 0)
        if tail:
            do_block(nrounds * weff, tail)

    return k


@functools.lru_cache(maxsize=None)
def _make_conv(E, V, N_pad):
    DH = 64
    per_t = E // NS
    nch = per_t // C
    nrounds = nch // W
    tail = nch - nrounds * W
    rows_ps = N_pad // NS
    reps = rows_ps // C

    scratch = ([pltpu.VMEM((per_t,), jnp.int32)]
               + [pltpu.VMEM((C, DH), F32) for _ in range(W)]
               + [pltpu.VMEM((C,), jnp.int32) for _ in range(W)]
               + [pltpu.VMEM((C, DH), F32)]
               + [pltpu.VMEM_SHARED((N_pad, DH), F32)]
               + [pltpu.SemaphoreType.DMA for _ in range(2 * W)])

    @functools.partial(
        pl.kernel,
        out_type=jax.ShapeDtypeStruct((N_pad, 128), F32),
        mesh=_mesh(),
        scratch_types=scratch,
        compiler_params=_cparams(DH),
    )
    def k(tabA, tabB, gi_hbm, gj_hbm, out, gi_all, v0, v1, v2, v3,
          i0, i1, i2, i3, zbuf, acc, s0, s1, s2, s3, t0, t1, t2, t3):
        vb = [v0, v1, v2, v3]
        ib = [i0, i1, i2, i3]
        vs = [s0, s1, s2, s3]
        isem = [t0, t1, t2, t3]
        cid = lax.axis_index("c")
        sid = lax.axis_index("s")
        col0 = cid * DH
        dpg = DH // 16

        def zb(t, _):
            zbuf[t // dpg, pl.ds((t % dpg) * 16, 16)] = jnp.zeros((16,), F32)
            return 0
        lax.fori_loop(0, C * dpg, zb, 0)

        def zc(r, _):
            pltpu.sync_copy(zbuf, acc.at[pl.ds(sid * rows_ps + r * C, C)])
            return 0
        lax.fori_loop(0, reps, zc, 0)
        plsc.subcore_barrier()

        base = sid * per_t
        pltpu.sync_copy(gi_hbm.at[pl.ds(base, per_t)], gi_all)

        def do_block(tab, c0, nw):
            gd = [pltpu.async_copy(
                tab.at[gi_all.at[pl.ds((c0 + w) * C, C)]],
                vb[w], vs[w]) for w in range(nw)]
            idd = [pltpu.async_copy(
                gj_hbm.at[pl.ds(base + (c0 + w) * C, C)],
                ib[w], isem[w]) for w in range(nw)]
            for w in range(nw):
                gd[w].wait()
                idd[w].wait()
                pltpu.sync_copy(vb[w], acc.at[ib[w]], add=True)

        def run(tab):
            if nrounds > 0:
                def round_body(r, _):
                    do_block(tab, r * W, W)
                    return 0
                lax.fori_loop(0, nrounds, round_body, 0)
            if tail:
                do_block(tab, nrounds * W, tail)

        @pl.when(cid == 0)
        def _():
            run(tabA)

        @pl.when(cid == 1)
        def _():
            run(tabB)

        plsc.subcore_barrier()
        pltpu.sync_copy(acc.at[pl.ds(sid * rows_ps, rows_ps)],
                        out.at[pl.ds(sid * rows_ps, rows_ps),
                               pl.ds(col0, DH)])

    return k


@functools.lru_cache(maxsize=None)
def _make_conv_narrow(E, V, N_pad):
    D = 16
    per_w = E // NW
    nch = per_w // C
    nrounds = nch // W
    tail = nch - nrounds * W
    rows_ps = N_pad // NS
    reps = rows_ps // C

    scratch = ([pltpu.VMEM((per_w,), jnp.int32)]
               + [pltpu.VMEM((C, D), F32) for _ in range(W)]
               + [pltpu.VMEM((C,), jnp.int32) for _ in range(W)]
               + [pltpu.VMEM((C, D), F32)]
               + [pltpu.VMEM_SHARED((N_pad, D), F32)]
               + [pltpu.SemaphoreType.DMA for _ in range(2 * W)])

    @functools.partial(
        pl.kernel,
        out_type=jax.ShapeDtypeStruct((NC * N_pad, D), F32),
        mesh=_mesh(),
        scratch_types=scratch,
        compiler_params=_cparams(D),
    )
    def k(table, gi_hbm, gj_hbm, out, gi_all, v0, v1, v2, v3,
          i0, i1, i2, i3, zbuf, acc, s0, s1, s2, s3, t0, t1, t2, t3):
        vb = [v0, v1, v2, v3]
        ib = [i0, i1, i2, i3]
        vs = [s0, s1, s2, s3]
        isem = [t0, t1, t2, t3]
        cid = lax.axis_index("c")
        sid = lax.axis_index("s")
        wid = sid * NC + cid

        def zb(t, _):
            zbuf[t, pl.ds(0, 16)] = jnp.zeros((16,), F32)
            return 0
        lax.fori_loop(0, C, zb, 0)

        def zc(r, _):
            pltpu.sync_copy(zbuf, acc.at[pl.ds(sid * rows_ps + r * C, C)])
            return 0
        lax.fori_loop(0, reps, zc, 0)
        plsc.subcore_barrier()

        base = wid * per_w
        pltpu.sync_copy(gi_hbm.at[pl.ds(base, per_w)], gi_all)

        def do_block(c0, nw):
            gd = [pltpu.async_copy(
                table.at[gi_all.at[pl.ds((c0 + w) * C, C)]],
                vb[w], vs[w]) for w in range(nw)]
            idd = [pltpu.async_copy(
                gj_hbm.at[pl.ds(base + (c0 + w) * C, C)],
                ib[w], isem[w]) for w in range(nw)]
            for w in range(nw):
                gd[w].wait()
                idd[w].wait()
                pltpu.sync_copy(vb[w], acc.at[ib[w]], add=True)

        if nrounds > 0:
            def round_body(r, _):
                do_block(r * W, W)
                return 0
            lax.fori_loop(0, nrounds, round_body, 0)
        if tail:
            do_block(nrounds * W, tail)

        plsc.subcore_barrier()
        pltpu.sync_copy(acc.at[pl.ds(sid * rows_ps, rows_ps)],
                        out.at[pl.ds(cid * N_pad + sid * rows_ps, rows_ps)])

    return k


@functools.lru_cache(maxsize=None)
def _make_scatter128(E, N_pad):
    DH = 64
    per_t = E // NS
    nch = per_t // C
    nrounds = nch // W
    tail = nch - nrounds * W
    rows_ps = N_pad // NS
    reps = rows_ps // C

    scratch = ([pltpu.VMEM((C, DH), F32) for _ in range(W)]
               + [pltpu.VMEM((C,), jnp.int32) for _ in range(W)]
               + [pltpu.VMEM((C, DH), F32)]
               + [pltpu.VMEM_SHARED((N_pad, DH), F32)]
               + [pltpu.SemaphoreType.DMA for _ in range(2 * W)])

    @functools.partial(
        pl.kernel,
        out_type=jax.ShapeDtypeStruct((N_pad, 128), F32),
        mesh=_mesh(),
        scratch_types=scratch,
        compiler_params=_cparams(DH),
    )
    def k(vals_hbm, idx_hbm, out, v0, v1, v2, v3, i0, i1, i2, i3,
          zbuf, acc, s0, s1, s2, s3, t0, t1, t2, t3):
        vb = [v0, v1, v2, v3]
        ib = [i0, i1, i2, i3]
        vs = [s0, s1, s2, s3]
        isem = [t0, t1, t2, t3]
        cid = lax.axis_index("c")
        sid = lax.axis_index("s")
        col0 = cid * DH
        dpg = DH // 16

        def zb(t, _):
            zbuf[t // dpg, pl.ds((t % dpg) * 16, 16)] = jnp.zeros((16,), F32)
            return 0
        lax.fori_loop(0, C * dpg, zb, 0)

        def zc(r, _):
            pltpu.sync_copy(zbuf, acc.at[pl.ds(sid * rows_ps + r * C, C)])
            return 0
        lax.fori_loop(0, reps, zc, 0)
        plsc.subcore_barrier()

        base = sid * per_t

        if nrounds > 0:
            def round_body(r, _):
                c0 = r * W
                vd = [pltpu.async_copy(
                    vals_hbm.at[pl.ds(base + (c0 + w) * C, C),
                                pl.ds(col0, DH)],
                    vb[w], vs[w]) for w in range(W)]
                idd = [pltpu.async_copy(
                    idx_hbm.at[pl.ds(base + (c0 + w) * C, C)],
                    ib[w], isem[w]) for w in range(W)]
                for w in range(W):
                    vd[w].wait()
                    idd[w].wait()
                    pltpu.sync_copy(vb[w], acc.at[ib[w]], add=True)
                return 0
            lax.fori_loop(0, nrounds, round_body, 0)
        for t in range(tail):
            c = nrounds * W + t
            pltpu.sync_copy(
                vals_hbm.at[pl.ds(base + c * C, C), pl.ds(col0, DH)], vb[0])
            pltpu.sync_copy(idx_hbm.at[pl.ds(base + c * C, C)], ib[0])
            pltpu.sync_copy(vb[0], acc.at[ib[0]], add=True)

        plsc.subcore_barrier()
        pltpu.sync_copy(acc.at[pl.ds(sid * rows_ps, rows_ps)],
                        out.at[pl.ds(sid * rows_ps, rows_ps),
                               pl.ds(col0, DH)])

    return k


@functools.lru_cache(maxsize=None)
def _make_scatter_narrow(E, N_pad, D):
    per_w = E // NW
    nch = per_w // C
    nrounds = nch // W
    tail = nch - nrounds * W
    rows_ps = N_pad // NS
    reps = rows_ps // C

    scratch = ([pltpu.VMEM((C, D), F32) for _ in range(W)]
               + [pltpu.VMEM((C,), jnp.int32) for _ in range(W)]
               + [pltpu.VMEM((C, D), F32)]
               + [pltpu.VMEM_SHARED((N_pad, D), F32)]
               + [pltpu.SemaphoreType.DMA for _ in range(2 * W)])

    @functools.partial(
        pl.kernel,
        out_type=jax.ShapeDtypeStruct((NC * N_pad, D), F32),
        mesh=_mesh(),
        scratch_types=scratch,
        compiler_params=_cparams(D),
    )
    def k(vals_hbm, idx_hbm, out, v0, v1, v2, v3, i0, i1, i2, i3,
          zbuf, acc, s0, s1, s2, s3, t0, t1, t2, t3):
        vb = [v0, v1, v2, v3]
        ib = [i0, i1, i2, i3]
        vs = [s0, s1, s2, s3]
        isem = [t0, t1, t2, t3]
        cid = lax.axis_index("c")
        sid = lax.axis_index("s")
        wid = sid * NC + cid
        dpg = D // 16

        def zb(t, _):
            zbuf[t // dpg, pl.ds((t % dpg) * 16, 16)] = jnp.zeros((16,), F32)
            return 0
        lax.fori_loop(0, C * dpg, zb, 0)

        def zc(r, _):
            pltpu.sync_copy(zbuf, acc.at[pl.ds(sid * rows_ps + r * C, C)])
            return 0
        lax.fori_loop(0, reps, zc, 0)
        plsc.subcore_barrier()

        base = wid * per_w

        if nrounds > 0:
            def round_body(r, _):
                c0 = r * W
                vd = [pltpu.async_copy(
                    vals_hbm.at[pl.ds(base + (c0 + w) * C, C)],
                    vb[w], vs[w]) for w in range(W)]
                idd = [pltpu.async_copy(
                    idx_hbm.at[pl.ds(base + (c0 + w) * C, C)],
                    ib[w], isem[w]) for w in range(W)]
                for w in range(W):
                    vd[w].wait()
                    idd[w].wait()
                    pltpu.sync_copy(vb[w], acc.at[ib[w]], add=True)
                return 0
            lax.fori_loop(0, nrounds, round_body, 0)
        for t in range(tail):
            c = nrounds * W + t
            pltpu.sync_copy(vals_hbm.at[pl.ds(base + c * C, C)], vb[0])
            pltpu.sync_copy(idx_hbm.at[pl.ds(base + c * C, C)], ib[0])
            pltpu.sync_copy(vb[0], acc.at[ib[0]], add=True)

        plsc.subcore_barrier()
        pltpu.sync_copy(acc.at[pl.ds(sid * rows_ps, rows_ps)],
                        out.at[pl.ds(cid * N_pad + sid * rows_ps, rows_ps)])

    return k


@functools.lru_cache(maxsize=None)
def _make_stats(E, N_pad):
    EC = 2000
    per_s = E // NS
    nch = per_s // EC
    rows_ps = N_pad // NS
    nvec = N_pad // 16
    nv = rows_ps // 16

    scratch = [
        pltpu.VMEM((N_pad,), F32),
        pltpu.VMEM((N_pad,), F32),
        pltpu.VMEM((EC,), jnp.int32),
        pltpu.VMEM((EC,), jnp.int32),
        pltpu.VMEM((NS, rows_ps), F32),
        pltpu.VMEM((rows_ps,), F32),
        pltpu.VMEM_SHARED((NS, N_pad), F32),
        pltpu.VMEM_SHARED((N_pad,), F32),
    ]

    @functools.partial(
        pl.kernel,
        out_type=(jax.ShapeDtypeStruct((N_pad,), F32),
                  jax.ShapeDtypeStruct((N_pad,), F32)),
        mesh=_mesh(),
        scratch_types=scratch,
        compiler_params=pltpu.CompilerParams(use_tc_tiling_on_sc=False,
                                             needs_layout_passes=False),
    )
    def k(ihbm, jhbm, invdeg_out, invagg_out,
          acc_t, inv_t, ib, jb, colbuf, slice_b, stage, degs):
        cid = lax.axis_index("c")
        sid = lax.axis_index("s")
        ones = jnp.ones((16,), F32)
        r0 = sid * rows_ps

        def zero_acc(t, _):
            acc_t[pl.ds(t * 16, 16)] = jnp.zeros((16,), F32)
            return 0

        def combine_to_slice():
            pltpu.sync_copy(acc_t, stage.at[sid])
            plsc.subcore_barrier()
            pltpu.sync_copy(stage.at[:, pl.ds(r0, rows_ps)], colbuf)

            def comb(v, _):
                s = jnp.zeros((16,), F32)
                for t in range(NS):
                    s = s + colbuf[t, pl.ds(v * 16, 16)]
                slice_b[pl.ds(v * 16, 16)] = s
                return 0
            lax.fori_loop(0, nv, comb, 0)

        lax.fori_loop(0, nvec, zero_acc, 0)

        def ch1(c, _):
            pltpu.sync_copy(ihbm.at[pl.ds(sid * per_s + c * EC, EC)], ib)

            def v1(v, _):
                iv = ib[pl.ds(v * 16, 16)]
                plsc.addupdate_scatter(acc_t, [iv], ones)
                return 0
            lax.fori_loop(0, EC // 16, v1, 0)
            return 0
        lax.fori_loop(0, nch, ch1, 0)

        combine_to_slice()
        pltpu.sync_copy(slice_b, degs.at[pl.ds(r0, rows_ps)])

        def invv(v, _):
            slice_b[pl.ds(v * 16, 16)] = 1.0 / slice_b[pl.ds(v * 16, 16)]
            return 0
        lax.fori_loop(0, nv, invv, 0)

        @pl.when(cid == 0)
        def _():
            pltpu.sync_copy(slice_b, invdeg_out.at[pl.ds(r0, rows_ps)])

        plsc.subcore_barrier()
        pltpu.sync_copy(degs, inv_t)

        def inv2(v, _):
            inv_t[pl.ds(v * 16, 16)] = 1.0 / inv_t[pl.ds(v * 16, 16)]
            return 0
        lax.fori_loop(0, nvec, inv2, 0)

        lax.fori_loop(0, nvec, zero_acc, 0)

        def ch2(c, _):
            pltpu.sync_copy(ihbm.at[pl.ds(sid * per_s + c * EC, EC)], ib)
            pltpu.sync_copy(jhbm.at[pl.ds(sid * per_s + c * EC, EC)], jb)

            def v2(v, _):
                iv = ib[pl.ds(v * 16, 16)]
                jv = jb[pl.ds(v * 16, 16)]
                g = plsc.load_gather(inv_t, [iv])
                plsc.addupdate_scatter(acc_t, [jv], g)
                return 0
            lax.fori_loop(0, EC // 16, v2, 0)
            return 0
        lax.fori_loop(0, nch, ch2, 0)

        combine_to_slice()

        def inva(v, _):
            slice_b[pl.ds(v * 16, 16)] = 1.0 / (slice_b[pl.ds(v * 16, 16)]
                                                + 1e-12)
            return 0
        lax.fori_loop(0, nv, inva, 0)

        @pl.when(cid == 0)
        def _():
            pltpu.sync_copy(slice_b, invagg_out.at[pl.ds(r0, rows_ps)])

    return k


def _full(shape):
    return pl.BlockSpec(shape, lambda i: tuple(0 for _ in shape))


def _rows(bs, d):
    return pl.BlockSpec((bs, d), lambda i: (i, 0))


def _edge_mlp(Pi, Qj, posi, posj, W0g, b0, W1, b1, W2, b2, g, beta):
    E = Pi.shape[0]
    BE = 512
    grid = E // BE

    def body(pi, qj, pa, pb, w0g, b0r, w1, b1r, w2, b2r, gr, br, out):
        d = pa[...] - pb[...]
        nrm = jnp.sqrt(jnp.sum(d * d, axis=1, keepdims=True))
        sel = (lax.broadcasted_iota(jnp.int32, (1, 16), 1) == 3).astype(F32)
        ge = d + nrm * sel
        s = pi[...].astype(F32) + qj[...].astype(F32)
        h0 = jnp.maximum(s + ge @ w0g[...] + b0r[...], 0.0)
        h1 = jnp.maximum(h0 @ w1[...] + b1r[...], 0.0)
        h2 = h1 @ w2[...] + b2r[...]
        mu = jnp.mean(h2, axis=1, keepdims=True)
        xc = h2 - mu
        var = jnp.mean(xc * xc, axis=1, keepdims=True)
        out[...] = xc / jnp.sqrt(var + 1e-5) * gr[...] + br[...]

    return pl.pallas_call(
        body,
        grid=(grid,),
        in_specs=[_rows(BE, 128), _rows(BE, 128), _rows(BE, 16),
                  _rows(BE, 16), _full((16, 128)), _full((1, 128)),
                  _full((128, 128)), _full((1, 128)), _full((128, 128)),
                  _full((1, 128)), _full((1, 128)), _full((1, 128))],
        out_specs=_rows(BE, 128),
        out_shape=jax.ShapeDtypeStruct((E, 128), F32),
    )(Pi, Qj, posi, posj, W0g, b0, W1, b1, W2, b2, g, beta)


def _node_mlp(x, aggr, A0, B0, b0, W1, b1, W2, b2, g, beta, res2=None):
    N = x.shape[0]
    BN = 2000 if N % 2000 == 0 else 512
    grid = N // BN

    def body(*refs):
        if res2 is None:
            (xr, ar, a0, b0m, b0r, w1, b1r, w2, b2r, gr, br, out) = refs
        else:
            (xr, ar, a0, b0m, b0r, w1, b1r, w2, b2r, gr, br, r2,
             out) = refs
        h0 = jnp.maximum(xr[...] @ a0[...] + ar[...] @ b0m[...] + b0r[...],
                         0.0)
        h1 = jnp.maximum(h0 @ w1[...] + b1r[...], 0.0)
        h2 = h1 @ w2[...] + b2r[...]
        mu = jnp.mean(h2, axis=1, keepdims=True)
        xc = h2 - mu
        var = jnp.mean(xc * xc, axis=1, keepdims=True)
        o = xc / jnp.sqrt(var + 1e-5) * gr[...] + br[...] + xr[...]
        if res2 is not None:
            o = o + r2[...]
        out[...] = o

    ins = [x, aggr, A0, B0, b0, W1, b1, W2, b2, g, beta]
    specs = [_rows(BN, 128), _rows(BN, 128),
             _full((128, 128)), _full((128, 128)), _full((1, 128)),
             _full((128, 128)), _full((1, 128)), _full((128, 128)),
             _full((1, 128)), _full((1, 128)), _full((1, 128))]
    if res2 is not None:
        ins.append(res2)
        specs.append(_rows(BN, 128))
    return pl.pallas_call(
        body,
        grid=(grid,),
        in_specs=specs,
        out_specs=_rows(BN, 128),
        out_shape=jax.ShapeDtypeStruct((N, 128), F32),
    )(*ins)


def _matmul2(x, Wa, Wb):
    N = x.shape[0]
    BN = 2000 if N % 2000 == 0 else 512
    grid = N // BN

    def body(xr, wa, wb, oa, ob):
        oa[...] = xr[...] @ wa[...]
        ob[...] = xr[...] @ wb[...]

    return pl.pallas_call(
        body,
        grid=(grid,),
        in_specs=[_rows(BN, 128), _full((128, 128)), _full((128, 128))],
        out_specs=[_rows(BN, 128), _rows(BN, 128)],
        out_shape=[jax.ShapeDtypeStruct((N, 128), F32),
                   jax.ShapeDtypeStruct((N, 128), F32)],
    )(x, Wa, Wb)


def _combine_scale(p0, p1, s, N, D):
    BN = 2000 if N % 2000 == 0 else 512
    grid = N // BN

    def body(a, b, sr, out):
        out[...] = (a[...] + b[...]) * sr[...]

    return pl.pallas_call(
        body,
        grid=(grid,),
        in_specs=[_rows(BN, D), _rows(BN, D), _rows(BN, 1)],
        out_specs=_rows(BN, D),
        out_shape=jax.ShapeDtypeStruct((N, D), F32),
    )(p0, p1, s)


def _scale(x, s):
    N, D = x.shape
    BN = 2000 if N % 2000 == 0 else 512
    grid = N // BN

    def body(a, sr, out):
        out[...] = a[...] * sr[...]

    return pl.pallas_call(
        body,
        grid=(grid,),
        in_specs=[_rows(BN, D), _rows(BN, 1)],
        out_specs=_rows(BN, D),
        out_shape=jax.ShapeDtypeStruct((N, D), F32),
    )(x, s)


def _scale_split(x, s):
    N, D = x.shape
    BN = 2000 if N % 2000 == 0 else 512
    grid = N // BN

    def body(a, sr, oa, ob):
        v = a[...] * sr[...]
        oa[...] = v[:, :64]
        ob[...] = v[:, 64:]

    return pl.pallas_call(
        body,
        grid=(grid,),
        in_specs=[_rows(BN, D), _rows(BN, 1)],
        out_specs=[_rows(BN, 64), _rows(BN, 64)],
        out_shape=[jax.ShapeDtypeStruct((N, 64), F32),
                   jax.ShapeDtypeStruct((N, 64), F32)],
    )(x, s)


def _split_edge_w(p):
    W0 = p['W0']
    W0g = jnp.zeros((16, 128), F32).at[:4].set(W0[:4])
    Wxi = W0[4:132]
    Wxj = W0[132:260]
    return (W0g, Wxi, Wxj, p['b0'].reshape(1, -1), p['W1'],
            p['b1'].reshape(1, -1), p['W2'], p['b2'].reshape(1, -1),
            p['g'].reshape(1, -1), p['beta'].reshape(1, -1))


def _split_node_w(p):
    W0 = p['W0']
    return (W0[:128], W0[128:], p['b0'].reshape(1, -1), p['W1'],
            p['b1'].reshape(1, -1), p['W2'], p['b2'].reshape(1, -1),
            p['g'].reshape(1, -1), p['beta'].reshape(1, -1))


def _gmp(x, posarg, gi, gj, gj_scat, pe, pn, N, E, N_pad, res2=None):
    W0g, Wxi, Wxj, b0, W1, b1, W2, b2, g, beta = _split_edge_w(pe)
    P, Q = _matmul2(x, Wxi, Wxj)
    V = x.shape[0]
    if isinstance(posarg, tuple):
        posi, posj = posarg
        mg = _make_multi_gather(((V, 128), (V, 128)), 2,
                                ((0, 0, 128, 'f32'), (1, 1, 128, 'f32')),
                                E)
        Pi, Qj = mg(P, Q, gi, gj)
    else:
        mg = _make_multi_gather(
            ((V, 128), (V, 128), (V, 16)), 2,
            ((0, 0, 128, 'f32'), (1, 1, 128, 'f32'),
             (2, 0, 16, 'f32'), (2, 1, 16, 'f32')), E)
        Pi, Qj, posi, posj = mg(P, Q, posarg, gi, gj)
    ee = _edge_mlp(Pi, Qj, posi, posj, W0g, b0, W1, b1, W2, b2, g, beta)
    aggr = _make_scatter128(E, N_pad)(ee, gj_scat)
    A0, B0, nb0, nW1, nb1, nW2, nb2, ng, nbeta = _split_node_w(pn)
    return _node_mlp(x[:N], aggr[:N], A0, B0, nb0,
                     nW1, nb1, nW2, nb2, ng, nbeta, res2=res2)


def kernel(h, pos, params, m_gs_0, m_gs_1, m_ids_0):
    N0, LD = h.shape
    E0 = m_gs_0.shape[1]
    N1 = m_ids_0.shape[0]
    E1 = m_gs_1.shape[1]
    N0p = 10240
    N1p = 2560
    E1p = 81920
    Bp = 2560

    i0 = m_gs_0[0]
    j0 = m_gs_0[1]
    pos_pad = jnp.pad(pos, ((0, 0), (0, 16 - pos.shape[1])))

    i1g = jnp.pad(m_gs_1[0], (0, E1p - E1))
    j1g = jnp.pad(m_gs_1[1], (0, E1p - E1))
    j1s = jnp.pad(m_gs_1[1], (0, E1p - E1), constant_values=N1p - 1)
    ids_g = jnp.pad(m_ids_0, (0, Bp - N1))
    ids_s = jnp.pad(m_ids_0, (0, Bp - N1), constant_values=N0p - 1)

    invdeg, invaggr = _make_stats(E0, N0p)(i0, j0)
    invdeg2 = invdeg[:N0].reshape(N0, 1)
    invaggr2 = invaggr[:N0].reshape(N0, 1)

    down = _make_multi_gather(((N0, 128), (N0, 128), (N0, 16)), 2,
                              ((0, 0, 128, 'f32'), (1, 1, 128, 'f32'),
                               (2, 0, 16, 'f32'), (2, 1, 16, 'f32')), E0)
    h1, posi0, posj0 = None, None, None
    W0g, Wxi, Wxj, b0, W1, b1, W2, b2, g, beta = _split_edge_w(
        params['down_e'])
    P, Q = _matmul2(h, Wxi, Wxj)
    Pi, Qj, posi0, posj0 = down(P, Q, pos_pad, i0, j0)
    ee = _edge_mlp(Pi, Qj, posi0, posj0, W0g, b0, W1, b1, W2, b2, g, beta)
    aggr = _make_scatter128(E0, N0p)(ee, j0)
    A0, B0, nb0, nW1, nb1, nW2, nb2, ng, nbeta = _split_node_w(
        params['down_n'])
    h1 = _node_mlp(h, aggr[:N0], A0, B0, nb0, nW1, nb1, nW2, nb2, ng, nbeta)

    conv0 = _make_conv(E0, N0, N0p)
    hhA, hhB = _scale_split(h1, invdeg2)
    h_conv = _scale(conv0(hhA, hhB, i0, j0)[:N0], invaggr2)

    pscaled = _scale(pos_pad, invdeg2)
    pparts = _make_conv_narrow(E0, N0, N0p)(pscaled, i0, j0)
    pos_conv = _combine_scale(pparts[:N0p], pparts[N0p:], invaggr2, N0, 16)

    h_pool, pos_pool = _make_multi_gather(
        ((N0, 128), (N0, 16)), 1, ((0, 0, 128, 'f32'), (1, 0, 16, 'f32')),
        Bp)(h_conv, pos_conv, ids_g)

    h_bot = _gmp(h_pool, pos_pool, i1g, j1g, j1s, params['bot_e'],
                 params['bot_n'], N1p, E1p, N1p)

    xup = _make_scatter128(Bp, N0p)(h_bot, ids_s)
    xupA, xupB = _scale_split(xup[:N0], invaggr2)
    h_up = _scale(conv0(xupA, xupB, j0, i0)[:N0], invdeg2)

    out = _gmp(h_up, (posi0, posj0), i0, j0, j0, params['up_e'],
               params['up_n'], N0, E0, N0p, res2=h1)
    return out
